# baseline (speedup 1.0000x reference)
import functools

import jax
import jax.numpy as jnp
from jax import lax
from jax.experimental import pallas as pl
from jax.experimental.pallas import tpu as pltpu
from jax.experimental.pallas import tpu_sc as plsc

NUM_FIELDS = 26
VOCAB = 100000
EMBED_DIM = 32
BATCH = 16384

_info = plsc.get_sparse_core_info()
NC, NS, L = _info.num_cores, _info.num_subcores, _info.num_lanes
NW = NC * NS
B_PER_W = BATCH // NW
CHUNK = 128
N_CHUNK = B_PER_W // CHUNK
IDX_ROWS = NUM_FIELDS * BATCH // CHUNK


def _body(idx_hbm, tab_hbm, out_hbm, idx_v, rows_v, sem):
    wid = lax.axis_index("s") * NC + lax.axis_index("c")

    @pl.loop(0, NUM_FIELDS)
    def _field(f):
        irow = f * (BATCH // CHUNK) + wid * N_CHUNK
        pltpu.sync_copy(idx_hbm.at[pl.ds(irow, N_CHUNK)], idx_v)
        off = f * VOCAB
        for j in range(N_CHUNK):
            for i in range(CHUNK // L):
                idx_v[j, pl.ds(i * L, L)] = idx_v[j, pl.ds(i * L, L)] + off
        copies = []
        for j in range(N_CHUNK):
            copies.append(
                pltpu.async_copy(
                    tab_hbm.at[idx_v.at[j]],
                    rows_v.at[pl.ds(j * CHUNK, CHUNK)],
                    sem,
                )
            )
        for c in copies:
            c.wait()
        obase = f * BATCH + wid * B_PER_W
        pltpu.sync_copy(rows_v, out_hbm.at[pl.ds(obase, B_PER_W)])


@jax.jit
def _encode(idx2d, tab_flat):
    mesh = plsc.VectorSubcoreMesh(core_axis_name="c", subcore_axis_name="s")
    return pl.kernel(
        _body,
        out_type=jax.ShapeDtypeStruct((NUM_FIELDS * BATCH, EMBED_DIM), jnp.float32),
        mesh=mesh,
        scratch_types=[
            pltpu.VMEM((N_CHUNK, CHUNK), jnp.int32),
            pltpu.VMEM((B_PER_W, EMBED_DIM), jnp.float32),
            pltpu.SemaphoreType.DMA,
        ],
        compiler_params=pltpu.CompilerParams(use_tc_tiling_on_sc=False),
    )(idx2d, tab_flat)


def kernel(inputs, tables):
    idx2d = inputs.T.reshape(IDX_ROWS, CHUNK).astype(jnp.int32)
    tab_flat = tables.reshape(NUM_FIELDS * VOCAB, EMBED_DIM)
    out = _encode(idx2d, tab_flat)
    out = out.reshape(NUM_FIELDS, BATCH, EMBED_DIM)
    return tuple(out[f] for f in range(NUM_FIELDS))


# speedup vs baseline: 1.2214x; 1.2214x over previous
import jax
import jax.numpy as jnp
from jax import lax
from jax.experimental import pallas as pl
from jax.experimental.pallas import tpu as pltpu
from jax.experimental.pallas import tpu_sc as plsc

NUM_FIELDS = 26
VOCAB = 100000
EMBED_DIM = 32
BATCH = 16384

_info = plsc.get_sparse_core_info()
NC, NS, L = _info.num_cores, _info.num_subcores, _info.num_lanes
NW = NC * NS
B_PER_W = BATCH // NW
CHUNK = 128
N_CHUNK = B_PER_W // CHUNK
IDX_ROWS = NUM_FIELDS * N_CHUNK


def _body(idx_hbm, tab_hbm, *refs):
    outs = refs[:NUM_FIELDS]
    idx_v, rows_a, rows_b, sem_a, sem_b, osem_a, osem_b = refs[NUM_FIELDS:]
    wid = lax.axis_index("s") * NC + lax.axis_index("c")

    pltpu.sync_copy(idx_hbm.at[wid], idx_v)

    @pl.loop(0, IDX_ROWS)
    def _add(j):
        off = (j // N_CHUNK) * VOCAB
        for i in range(CHUNK // L):
            idx_v[j, pl.ds(i * L, L)] = idx_v[j, pl.ds(i * L, L)] + off

    bufs = (rows_a, rows_b)
    sems = (sem_a, sem_b)
    osems = (osem_a, osem_b)
    obase = wid * B_PER_W
    gathers = [None, None]
    outcps = [None, None]

    for f in range(NUM_FIELDS):
        b = f & 1
        if outcps[b] is not None:
            outcps[b].wait()
        cps = []
        for j in range(N_CHUNK):
            cps.append(
                pltpu.async_copy(
                    tab_hbm.at[idx_v.at[f * N_CHUNK + j]],
                    bufs[b].at[pl.ds(j * CHUNK, CHUNK)],
                    sems[b],
                )
            )
        gathers[b] = cps
        if f >= 1:
            pb = 1 - b
            for c in gathers[pb]:
                c.wait()
            outcps[pb] = pltpu.async_copy(
                bufs[pb], outs[f - 1].at[pl.ds(obase, B_PER_W)], osems[pb]
            )
    lb = (NUM_FIELDS - 1) & 1
    for c in gathers[lb]:
        c.wait()
    outcps[lb] = pltpu.async_copy(
        bufs[lb], outs[NUM_FIELDS - 1].at[pl.ds(obase, B_PER_W)], osems[lb]
    )
    outcps[0].wait()
    outcps[1].wait()


@jax.jit
def _encode(idx3d, tab_flat):
    mesh = plsc.VectorSubcoreMesh(core_axis_name="c", subcore_axis_name="s")
    return pl.kernel(
        _body,
        out_type=tuple(
            jax.ShapeDtypeStruct((BATCH, EMBED_DIM), jnp.float32)
            for _ in range(NUM_FIELDS)
        ),
        mesh=mesh,
        scratch_types=[
            pltpu.VMEM((IDX_ROWS, CHUNK), jnp.int32),
            pltpu.VMEM((B_PER_W, EMBED_DIM), jnp.float32),
            pltpu.VMEM((B_PER_W, EMBED_DIM), jnp.float32),
            pltpu.SemaphoreType.DMA,
            pltpu.SemaphoreType.DMA,
            pltpu.SemaphoreType.DMA,
            pltpu.SemaphoreType.DMA,
        ],
        compiler_params=pltpu.CompilerParams(use_tc_tiling_on_sc=False),
    )(idx3d, tab_flat)


def kernel(inputs, tables):
    idx3d = (
        inputs.astype(jnp.int32)
        .reshape(NW, B_PER_W, NUM_FIELDS)
        .transpose(0, 2, 1)
        .reshape(NW, IDX_ROWS, CHUNK)
    )
    tab_flat = tables.reshape(NUM_FIELDS * VOCAB, EMBED_DIM)
    return _encode(idx3d, tab_flat)
